# baseline (speedup 1.0000x reference)
import functools

import jax
import jax.numpy as jnp
from jax import lax
from jax.experimental import pallas as pl
from jax.experimental.pallas import tpu as pltpu
from jax.experimental.pallas import tpu_sc as plsc

EMBED_DIM = 32
VOCAB = 1000000

NC = 2
NS = 16
NW = NC * NS

NB = 4096
NI = 200
B = NB * NI
IDX_ROW = 128
NSPLIT = 2
NI_S = NI // NSPLIT
B_S = NB * NI_S
KW = 4
CH = KW * IDX_ROW
B_PER_W = B_S // NW
NCHUNK = B_PER_W // CH
IDX_ROWS_W = B_PER_W // IDX_ROW
NT = NB // (4 * IDX_ROW)


T_BLK_V = 32768
T_GRID = -(-VOCAB // T_BLK_V)
V_PAD = T_GRID * T_BLK_V


def _relayout_body(wt_ref, out_ref):
    x = wt_ref[...]
    for q4 in range(T_BLK_V // 512):
        stacked = jnp.concatenate(
            [x[:, 512 * q4 + 128 * q:512 * q4 + 128 * (q + 1)]
             for q in range(4)], axis=0)
        out_ref[pl.ds(128 * q4, 128), :] = stacked.T


def _relayout(wt):
    return pl.pallas_call(
        _relayout_body,
        grid=(T_GRID,),
        in_specs=[pl.BlockSpec((EMBED_DIM, T_BLK_V), lambda i: (0, i))],
        out_specs=pl.BlockSpec((T_BLK_V // 4, 128), lambda i: (i, 0)),
        out_shape=jax.ShapeDtypeStruct((V_PAD // 4, 128), jnp.float32),
    )(wt)


def _body(table_hbm, idx_hbm, out_hbm,
          idx_v, rows_a, rows_b, gsa, gsb, osa, osb):
    cid = lax.axis_index("c")
    sid = lax.axis_index("s")
    wid = sid * NC + cid
    row0 = wid * IDX_ROWS_W
    unit0 = row0

    pltpu.sync_copy(idx_hbm.at[pl.ds(row0, IDX_ROWS_W)], idx_v)

    def fire_g(g, rows, gsem):
        for j in range(KW):
            pltpu.async_copy(
                table_hbm.at[idx_v.at[g * KW + j]],
                rows.at[pl.ds(j * IDX_ROW, IDX_ROW)],
                gsem,
            )

    def wait_g(rows, gsem):
        for j in range(KW):
            pltpu.make_async_copy(
                table_hbm.at[idx_v.at[j]],
                rows.at[pl.ds(j * IDX_ROW, IDX_ROW)],
                gsem,
            ).wait()

    def fire_s(g, rows, osem):
        for j in range(KW):
            u_num = unit0 + g * KW + j
            i = u_num // (4 * NT)
            t = lax.rem(u_num, 4 * NT) // 4
            pltpu.async_copy(
                rows.at[pl.ds(j * IDX_ROW, IDX_ROW)],
                out_hbm.at[i, t, :, pl.ds((j % 4) * EMBED_DIM, EMBED_DIM)],
                osem,
            )

    def wait_s(rows, osem):
        for j in range(KW):
            pltpu.make_async_copy(
                rows.at[pl.ds(j * IDX_ROW, IDX_ROW)],
                out_hbm.at[0, 0, :, pl.ds((j % 4) * EMBED_DIM, EMBED_DIM)],
                osem,
            ).wait()

    fire_g(0, rows_a, gsa)

    def chunk(g, carry):
        def half(rows_x, gsx, osx, rows_y, gsy, osy):
            wait_g(rows_x, gsx)

            @pl.when(g < NCHUNK - 1)
            def _():
                @pl.when(g >= 1)
                def _():
                    wait_s(rows_y, osy)

                fire_g(g + 1, rows_y, gsy)

            fire_s(g, rows_x, osx)

        even = lax.rem(g, 2) == 0
        pl.when(even)(lambda: half(rows_a, gsa, osa, rows_b, gsb, osb))
        pl.when(jnp.logical_not(even))(
            lambda: half(rows_b, gsb, osb, rows_a, gsa, osa))
        return carry

    lax.fori_loop(0, NCHUNK, chunk, 0)

    wait_s(rows_a, osa)
    wait_s(rows_b, osb)


def _lookup(idx2d, table):
    mesh = plsc.VectorSubcoreMesh(core_axis_name="c", subcore_axis_name="s")
    k = functools.partial(
        pl.kernel,
        out_type=jax.ShapeDtypeStruct((NI_S, NT, IDX_ROW, 4 * EMBED_DIM),
                                      jnp.float32),
        mesh=mesh,
        scratch_types=[
            pltpu.VMEM((IDX_ROWS_W, IDX_ROW), jnp.int32),
            pltpu.VMEM((CH, EMBED_DIM), jnp.float32),
            pltpu.VMEM((CH, EMBED_DIM), jnp.float32),
            pltpu.SemaphoreType.DMA,
            pltpu.SemaphoreType.DMA,
            pltpu.SemaphoreType.DMA,
            pltpu.SemaphoreType.DMA,
        ],
        compiler_params=pltpu.CompilerParams(use_tc_tiling_on_sc=False),
    )(_body)
    return k(table, idx2d)


O_BLK_I = 5
O_GRID_S = NI_S // O_BLK_I


def _out_tiles(w_ref, o_ref, i_off):
    for i in range(O_BLK_I):
        for t in range(NT):
            g = w_ref[i, t].T
            for dt in range(4):
                for u in range(4):
                    o_ref[i_off + i, dt, 4 * t + u, :, :] = \
                        g[32 * u + 8 * dt:32 * u + 8 * dt + 8, :]


def _out_body_a(w_ref, o_ref):
    _out_tiles(w_ref, o_ref, 0)


def _out_body_b(w_ref, prev_ref, o_ref):
    del prev_ref
    _out_tiles(w_ref, o_ref, 0)


OUT5_TY = jax.ShapeDtypeStruct((NI, 4, 4 * NT, 8, IDX_ROW), jnp.float32)
_W_SPEC = pl.BlockSpec((O_BLK_I, NT, IDX_ROW, IDX_ROW),
                       lambda i: (i, 0, 0, 0))


def _out_relayout_a(w):
    return pl.pallas_call(
        _out_body_a,
        grid=(O_GRID_S,),
        in_specs=[_W_SPEC],
        out_specs=pl.BlockSpec((O_BLK_I, 4, 4 * NT, 8, IDX_ROW),
                               lambda i: (i, 0, 0, 0, 0)),
        out_shape=OUT5_TY,
    )(w)


def _out_relayout_b(w, prev):
    return pl.pallas_call(
        _out_body_b,
        grid=(O_GRID_S,),
        in_specs=[_W_SPEC, pl.BlockSpec(memory_space=pl.ANY)],
        out_specs=pl.BlockSpec((O_BLK_I, 4, 4 * NT, 8, IDX_ROW),
                               lambda i: (i + O_GRID_S, 0, 0, 0, 0)),
        out_shape=OUT5_TY,
        input_output_aliases={1: 0},
    )(w, prev)


def kernel(inp, weight):
    v = jnp.asarray(inp, jnp.int32)
    row = (v & ~jnp.int32(511)) | ((v & 127) << 2) | ((v >> 7) & 3)
    idx2d = row.T.reshape(B // IDX_ROW, IDX_ROW)
    tab_lin = _relayout(weight.T)
    table = tab_lin.reshape(V_PAD, EMBED_DIM)
    half_rows = B_S // IDX_ROW
    staged1 = _lookup(idx2d[:half_rows], table)
    staged2 = _lookup(idx2d[half_rows:], table)
    o1 = _out_relayout_a(staged1)
    out5 = _out_relayout_b(staged2, o1)
    return out5.transpose(2, 4, 0, 1, 3).reshape(NB, NI, EMBED_DIM)


# speedup vs baseline: 1.0613x; 1.0613x over previous
import functools

import jax
import jax.numpy as jnp
from jax import lax
from jax.experimental import pallas as pl
from jax.experimental.pallas import tpu as pltpu
from jax.experimental.pallas import tpu_sc as plsc

EMBED_DIM = 32
VOCAB = 1000000

NC = 2
NS = 16
NW = NC * NS

NB = 4096
NI = 200
B = NB * NI
IDX_ROW = 128
CH = 1024
KW = CH // IDX_ROW
B_PER_W = B // NW
NCHUNK = B_PER_W // CH
IDX_ROWS_W = B_PER_W // IDX_ROW
NT = NB // (4 * IDX_ROW)


T_BLK_V = 65536
T_GRID = -(-VOCAB // T_BLK_V)
V_PAD = T_GRID * T_BLK_V


def _relayout_body(wt_ref, out_ref):
    x = wt_ref[...]
    for q4 in range(T_BLK_V // 512):
        stacked = jnp.concatenate(
            [x[:, 512 * q4 + 128 * q:512 * q4 + 128 * (q + 1)]
             for q in range(4)], axis=0)
        out_ref[pl.ds(128 * q4, 128), :] = stacked.T


def _relayout(wt):
    return pl.pallas_call(
        _relayout_body,
        grid=(T_GRID,),
        in_specs=[pl.BlockSpec((EMBED_DIM, T_BLK_V), lambda i: (0, i))],
        out_specs=pl.BlockSpec((T_BLK_V // 4, 128), lambda i: (i, 0)),
        out_shape=jax.ShapeDtypeStruct((V_PAD // 4, 128), jnp.float32),
    )(wt)


def _body(table_hbm, idx_hbm, out_hbm,
          idx_v, rows_a, rows_b, gsa, gsb, osa, osb):
    cid = lax.axis_index("c")
    sid = lax.axis_index("s")
    wid = sid * NC + cid
    row0 = wid * IDX_ROWS_W
    unit0 = row0

    pltpu.sync_copy(idx_hbm.at[pl.ds(row0, IDX_ROWS_W)], idx_v)

    def fire_g(g, rows, gsem):
        for j in range(KW):
            pltpu.async_copy(
                table_hbm.at[idx_v.at[g * KW + j]],
                rows.at[pl.ds(j * IDX_ROW, IDX_ROW)],
                gsem,
            )

    def wait_g(rows, gsem):
        for j in range(KW):
            pltpu.make_async_copy(
                table_hbm.at[idx_v.at[j]],
                rows.at[pl.ds(j * IDX_ROW, IDX_ROW)],
                gsem,
            ).wait()

    def fire_s(g, rows, osem):
        for j in range(KW):
            u_num = unit0 + g * KW + j
            i = u_num // (4 * NT)
            t = lax.rem(u_num, 4 * NT) // 4
            pltpu.async_copy(
                rows.at[pl.ds(j * IDX_ROW, IDX_ROW)],
                out_hbm.at[i, t, :, pl.ds((j % 4) * EMBED_DIM, EMBED_DIM)],
                osem,
            )

    def wait_s(rows, osem):
        for j in range(KW):
            pltpu.make_async_copy(
                rows.at[pl.ds(j * IDX_ROW, IDX_ROW)],
                out_hbm.at[0, 0, :, pl.ds((j % 4) * EMBED_DIM, EMBED_DIM)],
                osem,
            ).wait()

    fire_g(0, rows_a, gsa)

    def chunk(g, carry):
        def half(rows_x, gsx, osx, rows_y, gsy, osy):
            wait_g(rows_x, gsx)

            @pl.when(g < NCHUNK - 1)
            def _():
                @pl.when(g >= 1)
                def _():
                    wait_s(rows_y, osy)

                fire_g(g + 1, rows_y, gsy)

            fire_s(g, rows_x, osx)

        even = lax.rem(g, 2) == 0
        pl.when(even)(lambda: half(rows_a, gsa, osa, rows_b, gsb, osb))
        pl.when(jnp.logical_not(even))(
            lambda: half(rows_b, gsb, osb, rows_a, gsa, osa))
        return carry

    lax.fori_loop(0, NCHUNK, chunk, 0)

    wait_s(rows_a, osa)
    wait_s(rows_b, osb)


def _lookup(idx2d, table):
    mesh = plsc.VectorSubcoreMesh(core_axis_name="c", subcore_axis_name="s")
    k = functools.partial(
        pl.kernel,
        out_type=jax.ShapeDtypeStruct((NI, NT, IDX_ROW, 4 * EMBED_DIM),
                                      jnp.float32),
        mesh=mesh,
        scratch_types=[
            pltpu.VMEM((IDX_ROWS_W, IDX_ROW), jnp.int32),
            pltpu.VMEM((CH, EMBED_DIM), jnp.float32),
            pltpu.VMEM((CH, EMBED_DIM), jnp.float32),
            pltpu.SemaphoreType.DMA,
            pltpu.SemaphoreType.DMA,
            pltpu.SemaphoreType.DMA,
            pltpu.SemaphoreType.DMA,
        ],
        compiler_params=pltpu.CompilerParams(use_tc_tiling_on_sc=False),
    )(_body)
    return k(table, idx2d)


O_BLK_I = 20


def _out_body(w_ref, o_ref):
    for i in range(O_BLK_I):
        for t in range(NT):
            g = w_ref[i, t].T
            for dt in range(4):
                for u in range(4):
                    o_ref[i, dt, 4 * t + u, :, :] = \
                        g[32 * u + 8 * dt:32 * u + 8 * dt + 8, :]


def _out_relayout(w):
    return pl.pallas_call(
        _out_body,
        grid=(NI // O_BLK_I,),
        in_specs=[pl.BlockSpec((O_BLK_I, NT, IDX_ROW, IDX_ROW),
                               lambda i: (i, 0, 0, 0))],
        out_specs=pl.BlockSpec((O_BLK_I, 4, 4 * NT, 8, IDX_ROW),
                               lambda i: (i, 0, 0, 0, 0)),
        out_shape=jax.ShapeDtypeStruct((NI, 4, 4 * NT, 8, IDX_ROW),
                                       jnp.float32),
    )(w)


def kernel(inp, weight):
    v = jnp.asarray(inp, jnp.int32)
    row = (v & ~jnp.int32(511)) | ((v & 127) << 2) | ((v >> 7) & 3)
    idx2d = row.T.reshape(B // IDX_ROW, IDX_ROW)
    tab_lin = _relayout(weight.T)
    table = tab_lin.reshape(V_PAD, EMBED_DIM)
    staged = _lookup(idx2d, table)
    out5 = _out_relayout(staged)
    return out5.transpose(2, 4, 0, 1, 3).reshape(NB, NI, EMBED_DIM)
